# baseline (speedup 1.0000x reference)
import functools
import numpy as np
import jax
import jax.numpy as jnp
from jax import lax
from jax.experimental import pallas as pl
from jax.experimental.pallas import tpu as pltpu
from jax.experimental.pallas import tpu_sc as plsc

N, H, W, C = 4, 224, 224, 96
OH, OW = 2 * H, 2 * W
G = 8
GC = C // G
QV = (GC * W) // 16


def _sc_rows_body(nimg, img, mid, r0b, r1b, r2b, r3b,
                  oe0, oo0, oe1, oo1, sin0, sin1, sout0, sout1):
    ring = (r0b, r1b, r2b, r3b)
    obufs = ((oe0, oo0), (oe1, oo1))
    sins = (sin0, sin1)
    souts = (sout0, sout1)

    rc = 32 // (nimg * G)
    rows = H // rc

    wid = lax.axis_index("s") * 2 + lax.axis_index("c")
    n = wid // (G * rc)
    g = (wid // rc) % G
    r0 = (wid % rc) * rows
    c0 = g * GC

    def strip(r):
        rr = jnp.clip(r, 0, H - 1)
        return img.at[pl.ds(((n * H + rr) * C + c0) * W, GC * W)]

    def mrow(i):
        return mid.at[pl.ds(((n * OH + i) * C + c0) * W, GC * W)]

    pltpu.sync_copy(strip(r0 - 1), ring[3].at[...])
    pltpu.sync_copy(strip(r0), ring[0].at[...])
    pltpu.make_async_copy(strip(r0 + 1), ring[1].at[...], sins[1]).start()

    cw = jnp.float32(0.25)
    ch_ = jnp.float32(0.75)

    def do_t(lt, j):
        t = r0 + lt
        pslot = ring[(j - 1) % 4]
        cslot = ring[j % 4]
        nslot = ring[(j + 1) % 4]
        lslot = ring[(j + 2) % 4]

        pltpu.make_async_copy(strip(t + 2), lslot.at[...],
                              sins[(j + 2) % 2]).start()
        pltpu.make_async_copy(strip(t + 1), nslot.at[...],
                              sins[(j + 1) % 2]).wait()

        p = j % 2
        oe, oo = obufs[p]

        @pl.when(lt >= 2)
        def _():
            dummy = mid.at[pl.ds(0, GC * W)]
            pltpu.make_async_copy(dummy, oe.at[...], souts[p]).wait()
            pltpu.make_async_copy(dummy, oo.at[...], souts[p]).wait()

        def blend(q, _):
            o = q * 16
            pv = pslot[pl.ds(o, 16)]
            cv = cslot[pl.ds(o, 16)]
            nv = nslot[pl.ds(o, 16)]
            t1 = ch_ * cv
            oe[pl.ds(o, 16)] = cw * pv + t1
            oo[pl.ds(o, 16)] = t1 + cw * nv
            return 0

        lax.fori_loop(0, QV, blend, 0, unroll=4)

        pltpu.make_async_copy(oe.at[...], mrow(2 * t), souts[p]).start()
        pltpu.make_async_copy(oo.at[...], mrow(2 * t + 1), souts[p]).start()

    def tbody(i, _):
        for j in range(4):
            do_t(4 * i + j, j)
        return 0

    lax.fori_loop(0, rows // 4, tbody, 0)

    pltpu.make_async_copy(strip(r0), ring[1].at[...], sins[1]).wait()
    dummy = mid.at[pl.ds(0, GC * W)]
    for p in range(2):
        oe, oo = obufs[p]
        pltpu.make_async_copy(dummy, oe.at[...], souts[p]).wait()
        pltpu.make_async_copy(dummy, oo.at[...], souts[p]).wait()


def _sc_rows(imgt_flat, nimg):
    mesh = plsc.VectorSubcoreMesh(core_axis_name="c", subcore_axis_name="s")
    f = pl.kernel(
        functools.partial(_sc_rows_body, nimg),
        mesh=mesh,
        out_type=jax.ShapeDtypeStruct((nimg * OH * C * W,), jnp.float32),
        scratch_types=[pltpu.VMEM((GC * W,), jnp.float32)] * 8
        + [pltpu.SemaphoreType.DMA] * 4,
    )
    return f(imgt_flat)


def _col_matrix():
    a = np.zeros((W, 2 * W), np.float32)
    for m in range(W):
        a[max(m - 1, 0), 2 * m] += 0.25
        a[m, 2 * m] += 0.75
        a[m, 2 * m + 1] += 0.75
        a[min(m + 1, W - 1), 2 * m + 1] += 0.25
    return a.astype(jnp.bfloat16)


RB = 16


def _tc_cols_body(mid_ref, a_ref, out_ref):
    amat = a_ref[...]
    for r in range(RB):
        out_ref[0, r] = jax.lax.dot(
            mid_ref[0, r].astype(jnp.bfloat16), amat,
            preferred_element_type=jnp.float32)


def _tc_cols_body_acc(mid_ref, a_ref, prev_ref, out_ref):
    del prev_ref
    _tc_cols_body(mid_ref, a_ref, out_ref)


def _tc_cols_acc(mid4, prev_full, n):
    nblk = OH // RB
    if prev_full is None:
        body, specs, ops, aliases = (
            _tc_cols_body,
            [pl.BlockSpec((1, RB, C, W), lambda t: (0, t, 0, 0)),
             pl.BlockSpec((W, 2 * W), lambda t: (0, 0))],
            (mid4, _col_matrix()), {})
    else:
        body, specs, ops, aliases = (
            _tc_cols_body_acc,
            [pl.BlockSpec((1, RB, C, W), lambda t: (0, t, 0, 0)),
             pl.BlockSpec((W, 2 * W), lambda t: (0, 0)),
             pl.BlockSpec(memory_space=pltpu.MemorySpace.HBM)],
            (mid4, _col_matrix(), prev_full), {2: 0})
    return pl.pallas_call(
        body,
        grid=(nblk,),
        in_specs=specs,
        out_specs=pl.BlockSpec((1, RB, C, 2 * W),
                               lambda t, _n=n: (_n, t, 0, 0)),
        out_shape=jax.ShapeDtypeStruct((N, OH, C, 2 * W), jnp.float32),
        input_output_aliases=aliases,
        compiler_params=pltpu.CompilerParams(
            dimension_semantics=("arbitrary",)),
    )(*ops)


def kernel(img):
    imgt = img.transpose(0, 1, 3, 2)
    mids = [_sc_rows(imgt[n:n + 1].reshape(-1), 1) for n in range(N)]
    full = None
    for n in range(N):
        full = _tc_cols_acc(mids[n].reshape(1, OH, C, W), full, n)
    return full.reshape(N, OH, C, OW).transpose(0, 1, 3, 2)


# speedup vs baseline: 2.7072x; 2.7072x over previous
import functools
import numpy as np
import jax
import jax.numpy as jnp
from jax import lax
from jax.experimental import pallas as pl
from jax.experimental.pallas import tpu as pltpu
from jax.experimental.pallas import tpu_sc as plsc

N, H, W, C = 4, 224, 224, 96
OH, OW = 2 * H, 2 * W
NTC = 3
GC = 8
G = C // GC
RCH = 2
ROWS = H // RCH
TB = 16
RB = 16


def _col_matrix():
    a = np.zeros((W, 2 * W), np.float32)
    for m in range(W):
        a[max(m - 1, 0), 2 * m] += 0.25
        a[m, 2 * m] += 0.75
        a[m, 2 * m + 1] += 0.75
        a[min(m + 1, W - 1), 2 * m + 1] += 0.25
    return a.astype(jnp.bfloat16)


def _tc_single_body(prev_ref, mid_ref, next_ref, a_ref, out_ref):
    amat = a_ref[...]
    for r in range(TB):
        prow = mid_ref[0, r - 1] if r >= 1 else prev_ref[0, 0]
        crow = mid_ref[0, r]
        nrow = mid_ref[0, r + 1] if r < TB - 1 else next_ref[0, 0]
        for a, bl in ((0, 0.25 * prow + 0.75 * crow),
                      (1, 0.75 * crow + 0.25 * nrow)):
            out_ref[0, r, a] = jax.lax.dot(
                bl.astype(jnp.bfloat16), amat,
                preferred_element_type=jnp.float32)


def _tc_single(imgt):
    nblk = H // TB
    out5 = pl.pallas_call(
        _tc_single_body,
        grid=(NTC, nblk),
        in_specs=[
            pl.BlockSpec((1, 1, C, W),
                         lambda n, t: (n, jnp.maximum(t * TB - 1, 0), 0, 0)),
            pl.BlockSpec((1, TB, C, W), lambda n, t: (n, t, 0, 0)),
            pl.BlockSpec((1, 1, C, W),
                         lambda n, t: (n, jnp.minimum(t * TB + TB, H - 1), 0, 0)),
            pl.BlockSpec((W, 2 * W), lambda n, t: (0, 0)),
        ],
        out_specs=pl.BlockSpec((1, TB, 2, C, 2 * W),
                               lambda n, t: (n, t, 0, 0, 0)),
        out_shape=jax.ShapeDtypeStruct((N, H, 2, C, 2 * W), jnp.float32),
        compiler_params=pltpu.CompilerParams(
            dimension_semantics=("parallel", "arbitrary")),
    )(imgt, imgt, imgt, _col_matrix())
    return out5.reshape(N, OH, C, OW)


def _sc_rows_body(img, mid, r0b, r1b, r2b, r3b,
                  oe0, oo0, oe1, oo1, sin0, sin1, sout0, sout1):
    ring = (r0b, r1b, r2b, r3b)
    obufs = ((oe0, oo0), (oe1, oo1))
    sins = (sin0, sin1)
    souts = (sout0, sout1)

    wid = lax.axis_index("s") * 2 + lax.axis_index("c")

    @pl.when(wid < G * RCH)
    def _():
        g = wid // RCH
        r0 = (wid % RCH) * ROWS
        c0 = g * GC
        n = NTC

        def strip(r):
            rr = jnp.clip(r, 0, H - 1)
            return img.at[n, rr, pl.ds(c0, GC), :]

        def mrow(i):
            return mid.at[0, i, pl.ds(c0, GC), :]

        pltpu.sync_copy(strip(r0 - 1), ring[3].at[...])
        pltpu.sync_copy(strip(r0), ring[0].at[...])
        pltpu.make_async_copy(strip(r0 + 1), ring[1].at[...], sins[1]).start()

        cw = jnp.float32(0.25)
        ch_ = jnp.float32(0.75)

        def do_t(lt, j):
            t = r0 + lt
            pslot = ring[(j - 1) % 4]
            cslot = ring[j % 4]
            nslot = ring[(j + 1) % 4]
            lslot = ring[(j + 2) % 4]

            pltpu.make_async_copy(strip(t + 2), lslot.at[...],
                                  sins[(j + 2) % 2]).start()
            pltpu.make_async_copy(strip(t + 1), nslot.at[...],
                                  sins[(j + 1) % 2]).wait()

            p = j % 2
            oe, oo = obufs[p]

            @pl.when(lt >= 2)
            def _():
                dummy = mid.at[0, 0, pl.ds(0, GC), :]
                pltpu.make_async_copy(dummy, oe.at[...], souts[p]).wait()
                pltpu.make_async_copy(dummy, oo.at[...], souts[p]).wait()

            def blend(c, _):
                for q in range(W // 16):
                    o = q * 16
                    pv = pslot[c, pl.ds(o, 16)]
                    cv = cslot[c, pl.ds(o, 16)]
                    nv = nslot[c, pl.ds(o, 16)]
                    t1 = ch_ * cv
                    oe[c, pl.ds(o, 16)] = cw * pv + t1
                    oo[c, pl.ds(o, 16)] = t1 + cw * nv
                return 0

            lax.fori_loop(0, GC, blend, 0, unroll=2)

            pltpu.make_async_copy(oe.at[...], mrow(2 * t), souts[p]).start()
            pltpu.make_async_copy(oo.at[...], mrow(2 * t + 1), souts[p]).start()

        def tbody(i, _):
            for j in range(4):
                do_t(4 * i + j, j)
            return 0

        lax.fori_loop(0, ROWS // 4, tbody, 0)

        pltpu.make_async_copy(strip(r0), ring[1].at[...], sins[1]).wait()
        dummy = mid.at[0, 0, pl.ds(0, GC), :]
        for p in range(2):
            oe, oo = obufs[p]
            pltpu.make_async_copy(dummy, oe.at[...], souts[p]).wait()
            pltpu.make_async_copy(dummy, oo.at[...], souts[p]).wait()


def _sc_rows(imgt):
    mesh = plsc.VectorSubcoreMesh(core_axis_name="c", subcore_axis_name="s")
    f = pl.kernel(
        _sc_rows_body,
        mesh=mesh,
        out_type=jax.ShapeDtypeStruct((1, OH, C, W), jnp.float32),
        scratch_types=[pltpu.VMEM((GC, W), jnp.float32)] * 8
        + [pltpu.SemaphoreType.DMA] * 4,
    )
    return f(imgt)


def _tc_cols_body(mid_ref, a_ref, prev_ref, out_ref):
    del prev_ref
    amat = a_ref[...]
    for r in range(RB):
        out_ref[0, r] = jax.lax.dot(
            mid_ref[0, r].astype(jnp.bfloat16), amat,
            preferred_element_type=jnp.float32)


def _tc_cols(mid4, prev_full, n):
    nblk = OH // RB
    return pl.pallas_call(
        _tc_cols_body,
        grid=(nblk,),
        in_specs=[
            pl.BlockSpec((1, RB, C, W), lambda t: (0, t, 0, 0)),
            pl.BlockSpec((W, 2 * W), lambda t: (0, 0)),
            pl.BlockSpec(memory_space=pltpu.MemorySpace.HBM),
        ],
        out_specs=pl.BlockSpec((1, RB, C, 2 * W),
                               lambda t, _n=n: (_n, t, 0, 0)),
        out_shape=jax.ShapeDtypeStruct((N, OH, C, 2 * W), jnp.float32),
        input_output_aliases={2: 0},
        compiler_params=pltpu.CompilerParams(
            dimension_semantics=("arbitrary",)),
    )(mid4, _col_matrix(), prev_full)


def kernel(img):
    imgt = img.transpose(0, 1, 3, 2)
    mid3 = _sc_rows(imgt)
    full = _tc_single(imgt)
    full = _tc_cols(mid3, full.reshape(N, OH, C, 2 * W), NTC)
    return full.reshape(N, OH, C, OW).transpose(0, 1, 3, 2)


# speedup vs baseline: 3.3202x; 1.2264x over previous
import functools
import numpy as np
import jax
import jax.numpy as jnp
from jax import lax
from jax.experimental import pallas as pl
from jax.experimental.pallas import tpu as pltpu
from jax.experimental.pallas import tpu_sc as plsc

N, H, W, C = 4, 224, 224, 96
OH, OW = 2 * H, 2 * W
NTC = 3
GC = 8
G = C // GC
RCH = 2
SCR0 = H // 2
ROWS = (H - SCR0) // RCH
TB = 16
RB = 16
NBLK = H // TB
TSTEPS = NTC * NBLK + SCR0 // TB


def _col_matrix():
    a = np.zeros((W, 2 * W), np.float32)
    for m in range(W):
        a[max(m - 1, 0), 2 * m] += 0.25
        a[m, 2 * m] += 0.75
        a[m, 2 * m + 1] += 0.75
        a[min(m + 1, W - 1), 2 * m + 1] += 0.25
    return a.astype(jnp.bfloat16)


def _tc_single_body(prev_ref, mid_ref, next_ref, a_ref, out_ref):
    amat = a_ref[...]
    for r in range(TB):
        prow = mid_ref[0, r - 1] if r >= 1 else prev_ref[0, 0]
        crow = mid_ref[0, r]
        nrow = mid_ref[0, r + 1] if r < TB - 1 else next_ref[0, 0]
        for a, bl in ((0, 0.25 * prow + 0.75 * crow),
                      (1, 0.75 * crow + 0.25 * nrow)):
            out_ref[0, r, a] = jax.lax.dot(
                bl.astype(jnp.bfloat16), amat,
                preferred_element_type=jnp.float32)


def _tc_single(imgt):
    out5 = pl.pallas_call(
        _tc_single_body,
        grid=(TSTEPS,),
        in_specs=[
            pl.BlockSpec((1, 1, C, W),
                         lambda s: (s // NBLK,
                                    jnp.maximum((s % NBLK) * TB - 1, 0), 0, 0)),
            pl.BlockSpec((1, TB, C, W), lambda s: (s // NBLK, s % NBLK, 0, 0)),
            pl.BlockSpec((1, 1, C, W),
                         lambda s: (s // NBLK,
                                    jnp.minimum((s % NBLK) * TB + TB, H - 1),
                                    0, 0)),
            pl.BlockSpec((W, 2 * W), lambda s: (0, 0)),
        ],
        out_specs=pl.BlockSpec((1, TB, 2, C, 2 * W),
                               lambda s: (s // NBLK, s % NBLK, 0, 0, 0)),
        out_shape=jax.ShapeDtypeStruct((N, H, 2, C, 2 * W), jnp.float32),
        compiler_params=pltpu.CompilerParams(
            dimension_semantics=("arbitrary",)),
    )(imgt, imgt, imgt, _col_matrix())
    return out5.reshape(N, OH, C, OW)


def _sc_rows_body(img, mid, r0b, r1b, r2b, r3b,
                  oe0, oo0, oe1, oo1, sin0, sin1, sout0, sout1):
    ring = (r0b, r1b, r2b, r3b)
    obufs = ((oe0, oo0), (oe1, oo1))
    sins = (sin0, sin1)
    souts = (sout0, sout1)

    wid = lax.axis_index("s") * 2 + lax.axis_index("c")

    @pl.when(wid < G * RCH)
    def _():
        g = wid // RCH
        r0 = SCR0 + (wid % RCH) * ROWS
        c0 = g * GC
        n = NTC

        def strip(r):
            rr = jnp.clip(r, 0, H - 1)
            return img.at[n, rr, pl.ds(c0, GC), :]

        def mrow(i):
            return mid.at[0, i - 2 * SCR0, pl.ds(c0, GC), :]

        pltpu.sync_copy(strip(r0 - 1), ring[3].at[...])
        pltpu.sync_copy(strip(r0), ring[0].at[...])
        pltpu.make_async_copy(strip(r0 + 1), ring[1].at[...], sins[1]).start()

        cw = jnp.float32(0.25)
        ch_ = jnp.float32(0.75)

        def do_t(lt, j):
            t = r0 + lt
            pslot = ring[(j - 1) % 4]
            cslot = ring[j % 4]
            nslot = ring[(j + 1) % 4]
            lslot = ring[(j + 2) % 4]

            pltpu.make_async_copy(strip(t + 2), lslot.at[...],
                                  sins[(j + 2) % 2]).start()
            pltpu.make_async_copy(strip(t + 1), nslot.at[...],
                                  sins[(j + 1) % 2]).wait()

            p = j % 2
            oe, oo = obufs[p]

            @pl.when(lt >= 2)
            def _():
                dummy = mid.at[0, 0, pl.ds(0, GC), :]
                pltpu.make_async_copy(dummy, oe.at[...], souts[p]).wait()
                pltpu.make_async_copy(dummy, oo.at[...], souts[p]).wait()

            def blend(c, _):
                for q in range(W // 16):
                    o = q * 16
                    pv = pslot[c, pl.ds(o, 16)]
                    cv = cslot[c, pl.ds(o, 16)]
                    nv = nslot[c, pl.ds(o, 16)]
                    t1 = ch_ * cv
                    oe[c, pl.ds(o, 16)] = cw * pv + t1
                    oo[c, pl.ds(o, 16)] = t1 + cw * nv
                return 0

            lax.fori_loop(0, GC, blend, 0, unroll=2)

            pltpu.make_async_copy(oe.at[...], mrow(2 * t), souts[p]).start()
            pltpu.make_async_copy(oo.at[...], mrow(2 * t + 1), souts[p]).start()

        def tbody(i, _):
            for j in range(4):
                do_t(4 * i + j, j)
            return 0

        lax.fori_loop(0, ROWS // 4, tbody, 0)

        pltpu.make_async_copy(strip(r0), ring[1].at[...], sins[1]).wait()
        dummy = mid.at[0, 0, pl.ds(0, GC), :]
        for p in range(2):
            oe, oo = obufs[p]
            pltpu.make_async_copy(dummy, oe.at[...], souts[p]).wait()
            pltpu.make_async_copy(dummy, oo.at[...], souts[p]).wait()


def _sc_rows(imgt):
    mesh = plsc.VectorSubcoreMesh(core_axis_name="c", subcore_axis_name="s")
    f = pl.kernel(
        _sc_rows_body,
        mesh=mesh,
        out_type=jax.ShapeDtypeStruct((1, OH - 2 * SCR0, C, W), jnp.float32),
        scratch_types=[pltpu.VMEM((GC, W), jnp.float32)] * 8
        + [pltpu.SemaphoreType.DMA] * 4,
    )
    return f(imgt)


def _tc_cols_body(mid_ref, a_ref, prev_ref, out_ref):
    del prev_ref
    amat = a_ref[...]
    for r in range(RB):
        out_ref[0, r] = jax.lax.dot(
            mid_ref[0, r].astype(jnp.bfloat16), amat,
            preferred_element_type=jnp.float32)


def _tc_cols(mid4, prev_full, n):
    nblk = (OH - 2 * SCR0) // RB
    return pl.pallas_call(
        _tc_cols_body,
        grid=(nblk,),
        in_specs=[
            pl.BlockSpec((1, RB, C, W), lambda t: (0, t, 0, 0)),
            pl.BlockSpec((W, 2 * W), lambda t: (0, 0)),
            pl.BlockSpec(memory_space=pltpu.MemorySpace.HBM),
        ],
        out_specs=pl.BlockSpec((1, RB, C, 2 * W),
                               lambda t, _n=n: (_n, 2 * SCR0 // RB + t, 0, 0)),
        out_shape=jax.ShapeDtypeStruct((N, OH, C, 2 * W), jnp.float32),
        input_output_aliases={2: 0},
        compiler_params=pltpu.CompilerParams(
            dimension_semantics=("arbitrary",)),
    )(mid4, _col_matrix(), prev_full)


def kernel(img):
    imgt = img.transpose(0, 1, 3, 2)
    mid3 = _sc_rows(imgt)
    full = _tc_single(imgt)
    full = _tc_cols(mid3, full.reshape(N, OH, C, 2 * W), NTC)
    return full.reshape(N, OH, C, OW).transpose(0, 1, 3, 2)


# speedup vs baseline: 4.1566x; 1.2519x over previous
import numpy as np
import jax
import jax.numpy as jnp
from jax.experimental import pallas as pl
from jax.experimental.pallas import tpu as pltpu

N, H, W, C = 4, 224, 224, 96
TB = 16


def _col_matrix():
    a = np.zeros((W, 2 * W), np.float32)
    for m in range(W):
        a[max(m - 1, 0), 2 * m] += 0.25
        a[m, 2 * m] += 0.75
        a[m, 2 * m + 1] += 0.75
        a[min(m + 1, W - 1), 2 * m + 1] += 0.25
    return a.astype(jnp.bfloat16)


def _upsample_body(prev_ref, mid_ref, next_ref, a_ref, out_ref):
    amat = a_ref[...]
    for r in range(TB):
        prow = mid_ref[0, r - 1] if r >= 1 else prev_ref[0, 0]
        crow = mid_ref[0, r]
        nrow = mid_ref[0, r + 1] if r < TB - 1 else next_ref[0, 0]
        for a, bl in ((0, 0.25 * prow + 0.75 * crow),
                      (1, 0.75 * crow + 0.25 * nrow)):
            out_ref[0, r, a] = jax.lax.dot(
                bl.astype(jnp.bfloat16), amat,
                preferred_element_type=jnp.float32)


def kernel(img):
    imgt = img.transpose(0, 1, 3, 2)
    nblk = H // TB
    out5 = pl.pallas_call(
        _upsample_body,
        grid=(N, nblk),
        in_specs=[
            pl.BlockSpec((1, 1, C, W),
                         lambda n, t: (n, jnp.maximum(t * TB - 1, 0), 0, 0)),
            pl.BlockSpec((1, TB, C, W), lambda n, t: (n, t, 0, 0)),
            pl.BlockSpec((1, 1, C, W),
                         lambda n, t: (n, jnp.minimum(t * TB + TB, H - 1), 0, 0)),
            pl.BlockSpec((W, 2 * W), lambda n, t: (0, 0)),
        ],
        out_specs=pl.BlockSpec((1, TB, 2, C, 2 * W),
                               lambda n, t: (n, t, 0, 0, 0)),
        out_shape=jax.ShapeDtypeStruct((N, H, 2, C, 2 * W), img.dtype),
        compiler_params=pltpu.CompilerParams(
            dimension_semantics=("parallel", "arbitrary")),
    )(imgt, imgt, imgt, _col_matrix())
    return out5.reshape(N, 2 * H, C, 2 * W).transpose(0, 1, 3, 2)
